# baseline (speedup 1.0000x reference)
import functools

import jax
import jax.numpy as jnp
from jax import lax
from jax.experimental import pallas as pl
from jax.experimental.pallas import tpu as pltpu
from jax.experimental.pallas import tpu_sc as plsc

VOCAB = 1000000
EMB_DIM = 64
NUM_Y = 2
BATCH = 4096
HIST = 200

NUM_TILES = 32
BAGS_PER_TILE = BATCH // NUM_TILES
CHUNK = 104
OVERLAP = 2 * CHUNK - HIST
LANES = 16
NSLOTS = 8
MM_BLK = 16384


def _mm_body(w_ref, t_ref, p0_ref, p1_ref):
    res = lax.dot_general(w_ref[...], t_ref[...], (((1,), (0,)), ((), ())),
                          preferred_element_type=jnp.float32)
    p0_ref[...] = res[0]
    p1_ref[...] = res[1]


def _fold_classifier(w8, table_t):
    grid = (VOCAB + MM_BLK - 1) // MM_BLK
    out = pl.pallas_call(
        _mm_body,
        grid=(grid,),
        in_specs=[
            pl.BlockSpec((8, EMB_DIM), lambda i: (0, 0)),
            pl.BlockSpec((EMB_DIM, MM_BLK), lambda i: (0, i)),
        ],
        out_specs=[
            pl.BlockSpec((MM_BLK,), lambda i: (i,)),
            pl.BlockSpec((MM_BLK,), lambda i: (i,)),
        ],
        out_shape=[
            jax.ShapeDtypeStruct((VOCAB,), jnp.float32),
            jax.ShapeDtypeStruct((VOCAB,), jnp.float32),
        ],
    )(w8, table_t)
    return out[0], out[1]


def _sc_body(idx_hbm, p0_hbm, p1_hbm, b_hbm, out_hbm, idx_v, *rest):
    cbufs = rest[:2 * NSLOTS]
    b_v = rest[2 * NSLOTS]
    logit_v = rest[2 * NSLOTS + 1]
    sems = list(rest[2 * NSLOTS + 2:])
    bufs = [(cbufs[2 * i], cbufs[2 * i + 1]) for i in range(NSLOTS)]
    wid = lax.axis_index("s") * 2 + lax.axis_index("c")

    pltpu.sync_copy(idx_hbm.at[pl.ds(wid * BAGS_PER_TILE, BAGS_PER_TILE)],
                    idx_v)
    pltpu.sync_copy(b_hbm, b_v)

    b_reg = b_v[...]
    inv_n = jnp.float32(1.0 / HIST)
    lane_iota = lax.iota(jnp.int32, LANES)
    lane_mask = lane_iota < NUM_Y
    sub_mask = lane_iota < OVERLAP
    perms = [lane_iota ^ s for s in (8, 4, 2, 1)]
    fzero = jnp.zeros((LANES,), jnp.float32)

    def lane_sum(v):
        for p in perms:
            v = v + v.at[p].get(mode="promise_in_bounds")
        return v

    def fire(bag, slot):
        for p_hbm, cbuf in zip((p0_hbm, p1_hbm), bufs[slot]):
            pltpu.async_copy(p_hbm.at[idx_v.at[bag, pl.ds(0, CHUNK)]],
                             cbuf.at[pl.ds(0, CHUNK)], sems[slot])
            pltpu.async_copy(p_hbm.at[idx_v.at[bag, pl.ds(HIST - CHUNK,
                                                          CHUNK)]],
                             cbuf.at[pl.ds(CHUNK, CHUNK)], sems[slot])

    def drain(slot):
        for cbuf in bufs[slot]:
            for c in range(2):
                pltpu.make_async_copy(p0_hbm.at[pl.ds(0, CHUNK)],
                                      cbuf.at[pl.ds(c * CHUNK, CHUNK)],
                                      sems[slot]).wait()

    def class_sum(cbuf):
        s = cbuf[pl.ds(0, LANES)]
        for i in range(1, 2 * CHUNK // LANES):
            s = s + cbuf[pl.ds(i * LANES, LANES)]
        s = s - jnp.where(sub_mask, cbuf[pl.ds(CHUNK, LANES)], fzero)
        return lane_sum(s)

    def reduce_bag(bag, slot):
        tot0 = class_sum(bufs[slot][0])
        tot1 = class_sum(bufs[slot][1])
        x = jnp.where(lane_iota == 0, tot0, tot1) * inv_n + b_reg
        vals = 1.0 / (1.0 + jnp.exp(-x))
        plsc.store_scatter(logit_v, [2 * bag + lane_iota], vals,
                           mask=lane_mask)

    for i in range(NSLOTS - 1):
        fire(i, i)

    def group_body(g, carry):
        for u in range(NSLOTS):
            bag = NSLOTS * g + u
            drain(u)
            reduce_bag(bag, u)
            nxt = bag + NSLOTS - 1

            @pl.when(nxt < BAGS_PER_TILE)
            def _():
                fire(nxt, (u + NSLOTS - 1) % NSLOTS)
        return carry

    lax.fori_loop(0, BAGS_PER_TILE // NSLOTS, group_body, 0)

    pltpu.sync_copy(logit_v, out_hbm.at[pl.ds(wid * 2 * BAGS_PER_TILE,
                                              2 * BAGS_PER_TILE)])


@jax.jit
def _run(idx, table, w8, b_pad):
    p0, p1 = _fold_classifier(w8, table.T)
    sc = functools.partial(
        pl.kernel,
        out_type=jax.ShapeDtypeStruct((BATCH * NUM_Y,), jnp.float32),
        mesh=plsc.VectorSubcoreMesh(core_axis_name="c", subcore_axis_name="s"),
        compiler_params=pltpu.CompilerParams(
            needs_layout_passes=False, use_tc_tiling_on_sc=False),
        scratch_types=(
            [pltpu.VMEM((BAGS_PER_TILE, HIST), jnp.int32)]
            + [pltpu.VMEM((2 * CHUNK,), jnp.float32)
               for _ in range(2 * NSLOTS)]
            + [pltpu.VMEM((LANES,), jnp.float32),
               pltpu.VMEM((2 * BAGS_PER_TILE,), jnp.float32)]
            + [pltpu.SemaphoreType.DMA for _ in range(NSLOTS)]
        ),
    )(_sc_body)
    return sc(idx, p0, p1, b_pad)


def kernel(input, emb_weight, W, b):
    w8 = jnp.pad(W.astype(jnp.float32), ((0, 8 - NUM_Y), (0, 0)))
    b_pad = jnp.pad(b.astype(jnp.float32), (0, LANES - NUM_Y))
    out_flat = _run(input.astype(jnp.int32), emb_weight, w8, b_pad)
    return out_flat.reshape(BATCH, NUM_Y)


# speedup vs baseline: 1.0522x; 1.0522x over previous
import functools

import jax
import jax.numpy as jnp
from jax import lax
from jax.experimental import pallas as pl
from jax.experimental.pallas import tpu as pltpu
from jax.experimental.pallas import tpu_sc as plsc

VOCAB = 1000000
EMB_DIM = 64
NUM_Y = 2
BATCH = 4096
HIST = 200

NUM_TILES = 32
BAGS_PER_TILE = BATCH // NUM_TILES
CHUNK = 104
OVERLAP = 2 * CHUNK - HIST
LANES = 16
NSLOTS = 8
MM_BLK = 32768


def _mm_body(w_ref, t_ref, p0_ref, p1_ref):
    res = lax.dot_general(w_ref[...], t_ref[...], (((1,), (0,)), ((), ())),
                          preferred_element_type=jnp.float32)
    p0_ref[...] = res[0]
    p1_ref[...] = res[1]


def _fold_classifier(w8, table_t):
    grid = (VOCAB + MM_BLK - 1) // MM_BLK
    out = pl.pallas_call(
        _mm_body,
        grid=(grid,),
        in_specs=[
            pl.BlockSpec((8, EMB_DIM), lambda i: (0, 0)),
            pl.BlockSpec((EMB_DIM, MM_BLK), lambda i: (0, i)),
        ],
        out_specs=[
            pl.BlockSpec((MM_BLK,), lambda i: (i,)),
            pl.BlockSpec((MM_BLK,), lambda i: (i,)),
        ],
        out_shape=[
            jax.ShapeDtypeStruct((VOCAB,), jnp.float32),
            jax.ShapeDtypeStruct((VOCAB,), jnp.float32),
        ],
    )(w8, table_t)
    return out[0], out[1]


def _sc_body(idx_hbm, p0_hbm, p1_hbm, b_hbm, out_hbm, idx_v, *rest):
    cbufs = rest[:2 * NSLOTS]
    b_v = rest[2 * NSLOTS]
    logit_v = rest[2 * NSLOTS + 1]
    sems = list(rest[2 * NSLOTS + 2:])
    bufs = [(cbufs[2 * i], cbufs[2 * i + 1]) for i in range(NSLOTS)]
    wid = lax.axis_index("s") * 2 + lax.axis_index("c")

    pltpu.sync_copy(idx_hbm.at[pl.ds(wid * BAGS_PER_TILE, BAGS_PER_TILE)],
                    idx_v)
    pltpu.sync_copy(b_hbm, b_v)

    b_reg = b_v[...]
    inv_n = jnp.float32(1.0 / HIST)
    lane_iota = lax.iota(jnp.int32, LANES)
    lane_mask = lane_iota < NUM_Y
    sub_mask = lane_iota < OVERLAP
    perms = [lane_iota ^ s for s in (8, 4, 2, 1)]
    fzero = jnp.zeros((LANES,), jnp.float32)

    def lane_sum(v):
        for p in perms:
            v = v + v.at[p].get(mode="promise_in_bounds")
        return v

    def fire(bag, slot):
        for p_hbm, cbuf in zip((p0_hbm, p1_hbm), bufs[slot]):
            pltpu.async_copy(p_hbm.at[idx_v.at[bag, pl.ds(0, CHUNK)]],
                             cbuf.at[pl.ds(0, CHUNK)], sems[slot])
            pltpu.async_copy(p_hbm.at[idx_v.at[bag, pl.ds(HIST - CHUNK,
                                                          CHUNK)]],
                             cbuf.at[pl.ds(CHUNK, CHUNK)], sems[slot])

    def drain(slot):
        for cbuf in bufs[slot]:
            for c in range(2):
                pltpu.make_async_copy(p0_hbm.at[pl.ds(0, CHUNK)],
                                      cbuf.at[pl.ds(c * CHUNK, CHUNK)],
                                      sems[slot]).wait()

    def class_sum(cbuf):
        s = cbuf[pl.ds(0, LANES)]
        for i in range(1, 2 * CHUNK // LANES):
            s = s + cbuf[pl.ds(i * LANES, LANES)]
        s = s - jnp.where(sub_mask, cbuf[pl.ds(CHUNK, LANES)], fzero)
        return lane_sum(s)

    def reduce_bag(bag, slot):
        tot0 = class_sum(bufs[slot][0])
        tot1 = class_sum(bufs[slot][1])
        x = jnp.where(lane_iota == 0, tot0, tot1) * inv_n + b_reg
        vals = 1.0 / (1.0 + jnp.exp(-x))
        plsc.store_scatter(logit_v, [2 * bag + lane_iota], vals,
                           mask=lane_mask)

    for i in range(NSLOTS - 1):
        fire(i, i)

    def group_body(g, carry):
        for u in range(NSLOTS):
            bag = NSLOTS * g + u
            drain(u)
            reduce_bag(bag, u)
            nxt = bag + NSLOTS - 1

            @pl.when(nxt < BAGS_PER_TILE)
            def _():
                fire(nxt, (u + NSLOTS - 1) % NSLOTS)
        return carry

    lax.fori_loop(0, BAGS_PER_TILE // NSLOTS, group_body, 0)

    pltpu.sync_copy(logit_v, out_hbm.at[pl.ds(wid * 2 * BAGS_PER_TILE,
                                              2 * BAGS_PER_TILE)])


@jax.jit
def _run(idx, table, w8, b_pad):
    p0, p1 = _fold_classifier(w8, table.T)
    sc = functools.partial(
        pl.kernel,
        out_type=jax.ShapeDtypeStruct((BATCH * NUM_Y,), jnp.float32),
        mesh=plsc.VectorSubcoreMesh(core_axis_name="c", subcore_axis_name="s"),
        compiler_params=pltpu.CompilerParams(
            needs_layout_passes=False, use_tc_tiling_on_sc=False),
        scratch_types=(
            [pltpu.VMEM((BAGS_PER_TILE, HIST), jnp.int32)]
            + [pltpu.VMEM((2 * CHUNK,), jnp.float32)
               for _ in range(2 * NSLOTS)]
            + [pltpu.VMEM((LANES,), jnp.float32),
               pltpu.VMEM((2 * BAGS_PER_TILE,), jnp.float32)]
            + [pltpu.SemaphoreType.DMA for _ in range(NSLOTS)]
        ),
    )(_sc_body)
    return sc(idx, p0, p1, b_pad)


def kernel(input, emb_weight, W, b):
    w8 = jnp.pad(W.astype(jnp.float32), ((0, 8 - NUM_Y), (0, 0)))
    b_pad = jnp.pad(b.astype(jnp.float32), (0, LANES - NUM_Y))
    out_flat = _run(input.astype(jnp.int32), emb_weight, w8, b_pad)
    return out_flat.reshape(BATCH, NUM_Y)
